# baseline (speedup 1.0000x reference)
import functools

import jax
import jax.numpy as jnp
from jax import lax
from jax.experimental import pallas as pl
from jax.experimental.pallas import tpu as pltpu
from jax.experimental.pallas import tpu_sc as plsc

DIM = 64
BATCH = 16384
LANES = 16
NUM_CORES = 2
NUM_SUBCORES = 16
NUM_WORKERS = NUM_CORES * NUM_SUBCORES
ROWS_PER_W = BATCH // NUM_WORKERS
CHUNK = 128
NUM_CHUNKS = ROWS_PER_W // CHUNK
NUM_GROUPS = DIM // LANES


def _score_kernel(h_hbm, r_hbm, t_hbm, ere_hbm, eim_hbm, rre_hbm, rim_hbm,
                  out_hbm,
                  hidx_v, ridx_v, tidx_v,
                  hr_v, hi_v, tr_v, ti_v, rr_v, ri_v,
                  out_v, sem):
    wid = lax.axis_index("s") * NUM_CORES + lax.axis_index("c")
    base = wid * ROWS_PER_W

    pltpu.sync_copy(h_hbm.at[pl.ds(base, ROWS_PER_W)], hidx_v)
    pltpu.sync_copy(r_hbm.at[pl.ds(base, ROWS_PER_W)], ridx_v)
    pltpu.sync_copy(t_hbm.at[pl.ds(base, ROWS_PER_W)], tidx_v)

    for c in range(NUM_CHUNKS):
        off = c * CHUNK
        hsl = hidx_v.at[pl.ds(off, CHUNK)]
        rsl = ridx_v.at[pl.ds(off, CHUNK)]
        tsl = tidx_v.at[pl.ds(off, CHUNK)]
        copies = [
            pltpu.async_copy(ere_hbm.at[hsl], hr_v, sem),
            pltpu.async_copy(eim_hbm.at[hsl], hi_v, sem),
            pltpu.async_copy(ere_hbm.at[tsl], tr_v, sem),
            pltpu.async_copy(eim_hbm.at[tsl], ti_v, sem),
            pltpu.async_copy(rre_hbm.at[rsl], rr_v, sem),
            pltpu.async_copy(rim_hbm.at[rsl], ri_v, sem),
        ]
        for cp in copies:
            cp.wait()

        lane_iota = lax.iota(jnp.int32, LANES)

        def group_body(g16, _, off=off):
            rows = g16 * LANES + lane_iota

            def dim_body(d, acc):
                cols = jnp.full((LANES,), 0, jnp.int32) + d
                idx = [rows, cols]
                hr = plsc.load_gather(hr_v, idx)
                hi = plsc.load_gather(hi_v, idx)
                tr = plsc.load_gather(tr_v, idx)
                ti = plsc.load_gather(ti_v, idx)
                rr = plsc.load_gather(rr_v, idx)
                ri = plsc.load_gather(ri_v, idx)
                return (acc + rr * (hr * tr + hi * ti)
                        + ri * (hr * ti - hi * tr))

            scores = lax.fori_loop(0, DIM, dim_body,
                                   jnp.zeros((LANES,), jnp.float32))
            out_v[pl.ds(off + g16 * LANES, LANES)] = scores
            return 0

        lax.fori_loop(0, CHUNK // LANES, group_body, 0)

    pltpu.sync_copy(out_v, out_hbm.at[pl.ds(base, ROWS_PER_W)])


@functools.partial(jax.jit)
def _score(h, r, t, entity_re, entity_im, rel_re, rel_im):
    mesh = plsc.VectorSubcoreMesh(core_axis_name="c", subcore_axis_name="s")
    kern = functools.partial(
        pl.kernel,
        mesh=mesh,
        out_type=jax.ShapeDtypeStruct((BATCH,), jnp.float32),
        compiler_params=pltpu.CompilerParams(
            needs_layout_passes=False, use_tc_tiling_on_sc=False),
        scratch_types=[
            pltpu.VMEM((ROWS_PER_W,), jnp.int32),
            pltpu.VMEM((ROWS_PER_W,), jnp.int32),
            pltpu.VMEM((ROWS_PER_W,), jnp.int32),
            pltpu.VMEM((CHUNK, DIM), jnp.float32),
            pltpu.VMEM((CHUNK, DIM), jnp.float32),
            pltpu.VMEM((CHUNK, DIM), jnp.float32),
            pltpu.VMEM((CHUNK, DIM), jnp.float32),
            pltpu.VMEM((CHUNK, DIM), jnp.float32),
            pltpu.VMEM((CHUNK, DIM), jnp.float32),
            pltpu.VMEM((ROWS_PER_W,), jnp.float32),
            pltpu.SemaphoreType.DMA,
        ],
    )(_score_kernel)
    return kern(h, r, t, entity_re, entity_im, rel_re, rel_im)


def kernel(h, r, t, entity_re, entity_im, rel_re, rel_im):
    return _score(h, r, t, entity_re, entity_im, rel_re, rel_im)
